# Initial kernel scaffold; baseline (speedup 1.0000x reference)
import jax
import jax.numpy as jnp
from jax.experimental import pallas as pl


def kernel(four_momenta, group1_indices, group2_indices):
    raise NotImplementedError("write your pallas kernel here")


# speedup vs baseline: 1.4943x; 1.4943x over previous
import functools
import itertools

import jax
import jax.numpy as jnp
from jax import lax
from jax.experimental import pallas as pl
from jax.experimental.pallas import tpu as pltpu
from jax.experimental.pallas import tpu_sc as plsc

_LANES = 16
_TRIPLETS = tuple(itertools.combinations(range(7), 3))
_NT = len(_TRIPLETS)


def _sqrt_pos(x):
    i = plsc.bitcast(x, jnp.int32)
    y = plsc.bitcast(jnp.int32(0x5F3759DF) - lax.shift_right_arithmetic(i, 1),
                     jnp.float32)
    xh = 0.5 * x
    y = y * (1.5 - xh * y * y)
    y = y * (1.5 - xh * y * y)
    return x * y


def _ranks16(g_ref, k, iota, na):
    rows = jnp.minimum(iota + jnp.int32(16 * k), jnp.int32(na - 1)) * 3
    a = plsc.load_gather(g_ref, [rows])
    b = plsc.load_gather(g_ref, [rows + 1])
    c = plsc.load_gather(g_ref, [rows + 2])
    t = (7 - a) * (6 - a) * (5 - a)
    s1 = 35 - lax.shift_right_logical(t * 683, 12)
    s2 = lax.shift_right_logical((b - a - 1) * (12 - a - b), 1)
    s3 = c - b - 1
    return (s1 + s2 + s3) * 16


@functools.lru_cache(maxsize=None)
def _build(batch, na):
    nw = 32
    rw = batch // nw
    ng = rw // _LANES
    nav = -(-na // _LANES)
    mesh = plsc.VectorSubcoreMesh(core_axis_name="c", subcore_axis_name="s")

    @functools.partial(
        pl.kernel,
        mesh=mesh,
        out_type=jax.ShapeDtypeStruct((batch * na,), jnp.float32),
        compiler_params=pltpu.CompilerParams(needs_layout_passes=False),
        scratch_types=[
            pltpu.VMEM((rw * 28,), jnp.float32),
            pltpu.VMEM((na * 3,), jnp.int32),
            pltpu.VMEM((na * 3,), jnp.int32),
            pltpu.VMEM((_NT * _LANES,), jnp.float32),
            pltpu.VMEM((rw * na,), jnp.float32),
        ],
    )
    def sc_kernel(fm_hbm, g1_hbm, g2_hbm, out_hbm,
                  fm_v, g1_v, g2_v, mass_v, out_v):
        wid = lax.axis_index("s") * 2 + lax.axis_index("c")
        pltpu.sync_copy(fm_hbm.at[pl.ds(wid * (rw * 28), rw * 28)], fm_v)
        pltpu.sync_copy(g1_hbm, g1_v)
        pltpu.sync_copy(g2_hbm, g2_v)

        iota = jnp.arange(_LANES, dtype=jnp.int32)
        r1_vecs = [_ranks16(g1_v, k, iota, na) for k in range(nav)]
        r2_vecs = [_ranks16(g2_v, k, iota, na) for k in range(nav)]
        off1s = [r1_vecs[a // _LANES][a % _LANES] for a in range(na)]
        off2s = [r2_vecs[a // _LANES][a % _LANES] for a in range(na)]
        iota28 = iota * 28
        iota_na = iota * na

        def group_body(g, carry):
            rows28 = iota28 + g * (28 * _LANES)
            jets = [[plsc.load_gather(fm_v, [rows28 + (4 * j + c)])
                     for c in range(4)] for j in range(7)]
            pair_cache = {}

            def pair_sum(i, j):
                if (i, j) not in pair_cache:
                    pair_cache[(i, j)] = [jets[i][c] + jets[j][c]
                                          for c in range(4)]
                return pair_cache[(i, j)]

            for t, (i, j, k) in enumerate(_TRIPLETS):
                p = pair_sum(i, j)
                e = p[0] + jets[k][0]
                x = p[1] + jets[k][1]
                y = p[2] + jets[k][2]
                z = p[3] + jets[k][3]
                m2 = e * e - x * x - y * y - z * z
                mass_v[pl.ds(t * _LANES, _LANES)] = _sqrt_pos(
                    jnp.maximum(m2, 1e-12))

            rows_na = iota_na + g * (na * _LANES)
            for a in range(na):
                m1 = mass_v[pl.ds(off1s[a], _LANES)]
                m2m = mass_v[pl.ds(off2s[a], _LANES)]
                num = jnp.abs(m1 - m2m)
                den = jnp.maximum(m1 + m2m, 1e-8)
                plsc.store_scatter(out_v, [rows_na + a], -(num / den))
            return carry

        lax.fori_loop(0, ng, group_body, jnp.int32(0))
        pltpu.sync_copy(out_v, out_hbm.at[pl.ds(wid * (rw * na), rw * na)])

    return sc_kernel


def kernel(four_momenta, group1_indices, group2_indices):
    batch, nj, nc = four_momenta.shape
    na = group1_indices.shape[0]
    fm_flat = four_momenta.reshape(batch * nj * nc)
    out = _build(batch, na)(fm_flat,
                            group1_indices.astype(jnp.int32).reshape(na * 3),
                            group2_indices.astype(jnp.int32).reshape(na * 3))
    return out.reshape(batch, na)
